# Initial kernel scaffold; baseline (speedup 1.0000x reference)
import jax
import jax.numpy as jnp
from jax.experimental import pallas as pl


def kernel(cons_embedding, vals_embedding, v2c_edge_index, c2v_edge_index, v2c_edge_attr, c2v_edge_attr, cons_batch, vals_batch, v2c_We, v2c_be, v2c_W1, v2c_b1, v2c_W2, v2c_b2, c2v_We, c2v_be, c2v_W1, c2v_b1, c2v_W2, c2v_b2):
    raise NotImplementedError("write your pallas kernel here")


# speedup vs baseline: 4.5900x; 4.5900x over previous
import functools

import jax
import jax.numpy as jnp
from jax import lax
from jax.experimental import pallas as pl
from jax.experimental.pallas import tpu as pltpu
from jax.experimental.pallas import tpu_sc as plsc

N_NODE = 10000
E = 320000
D = 128
D_EDGE = 16
D_HID = 256

NC = 2
NS = 16
NW = NC * NS
EPW = E // NW
CHUNK = 80
NCHUNK = EPW // CHUNK
DRAIN_TILES = 10
DRAIN_ROWS = N_NODE // DRAIN_TILES
ZROWS = 40
LANES = 16


def _sc_gather_scatter(x_src, row_idx, col_idx, q):
  mesh = plsc.VectorSubcoreMesh(core_axis_name="c", subcore_axis_name="s",
                                num_cores=NC, num_subcores=NS)

  @functools.partial(
      pl.kernel,
      out_type=jax.ShapeDtypeStruct((NC, N_NODE, D), jnp.float32),
      mesh=mesh,
      scratch_types=[
          pltpu.VMEM((3, CHUNK), jnp.int32),
          pltpu.VMEM((3, CHUNK), jnp.int32),
          pltpu.VMEM((2, CHUNK, D), jnp.float32),
          pltpu.VMEM((2, CHUNK, D), jnp.float32),
          pltpu.VMEM_SHARED((N_NODE, D), jnp.float32),
          pltpu.SemaphoreType.DMA((3,)),
          pltpu.SemaphoreType.DMA((2,)),
          pltpu.SemaphoreType.DMA((2,)),
      ],
  )
  def conv(x_hbm, row_hbm, col_hbm, q_hbm, out_hbm, rowv, colv, qv, rv,
           agg, sidx, sem, sscat):
    c = lax.axis_index("c")
    s = lax.axis_index("s")
    wid = c * NS + s

    zero = jnp.zeros((LANES,), jnp.float32)

    def zbody(r, carry):
      for j in range(D // LANES):
        rv[0, r, pl.ds(j * LANES, LANES)] = zero
      return carry

    lax.fori_loop(0, ZROWS, zbody, 0)

    @pl.when(s < DRAIN_TILES)
    def _zero():
      for k in range(DRAIN_ROWS // ZROWS):
        pltpu.sync_copy(rv.at[0, pl.ds(0, ZROWS)],
                        agg.at[pl.ds(s * DRAIN_ROWS + k * ZROWS, ZROWS)])

    plsc.subcore_barrier()

    def fetch_idx(i, slot):
      base = wid * EPW + i * CHUNK
      pltpu.async_copy(row_hbm.at[pl.ds(base, CHUNK)], rowv.at[slot],
                       sidx.at[slot])
      pltpu.async_copy(col_hbm.at[pl.ds(base, CHUNK)], colv.at[slot],
                       sidx.at[slot])

    def wait_idx(slot):
      d = pltpu.make_async_copy(row_hbm.at[pl.ds(0, CHUNK)], rowv.at[slot],
                                sidx.at[slot])
      d.wait()
      d.wait()

    def fetch_data(i, islot, dslot):
      base = wid * EPW + i * CHUNK
      pltpu.async_copy(q_hbm.at[pl.ds(base, CHUNK)], qv.at[dslot],
                       sem.at[dslot])
      pltpu.async_copy(x_hbm.at[rowv.at[islot]], rv.at[dslot], sem.at[dslot])

    def wait_data(dslot):
      d = pltpu.make_async_copy(q_hbm.at[pl.ds(0, CHUNK)], qv.at[dslot],
                                sem.at[dslot])
      d.wait()
      d.wait()

    def wait_scat(dslot):
      pltpu.make_async_copy(rv.at[dslot], agg.at[pl.ds(0, CHUNK)],
                            sscat.at[dslot]).wait()

    fetch_idx(0, 0)
    fetch_idx(1, 1)
    wait_idx(0)
    fetch_data(0, 0, 0)

    def body(i, carry):
      i3 = lax.rem(i, 3)
      n3 = lax.rem(i + 1, 3)
      p3 = lax.rem(i + 2, 3)
      cur = lax.rem(i, 2)
      nxt = lax.rem(i + 1, 2)

      @pl.when(i + 1 < NCHUNK)
      def _launch_next():
        wait_idx(n3)

        @pl.when(i >= 1)
        def _wait_prev_scatter():
          wait_scat(nxt)

        fetch_data(i + 1, n3, nxt)

      @pl.when(i + 2 < NCHUNK)
      def _prefetch_idx():
        fetch_idx(i + 2, p3)

      wait_data(cur)

      @plsc.parallel_loop(0, CHUNK, 1, unroll=8)
      def cbody(e):
        for j in range(D // LANES):
          sl = pl.ds(j * LANES, LANES)
          rv[cur, e, sl] = jnp.maximum(rv[cur, e, sl] + qv[cur, e, sl], 0.0)

      pltpu.async_copy(rv.at[cur], agg.at[colv.at[i3]], sscat.at[cur],
                       add=True)
      return carry

    lax.fori_loop(0, NCHUNK, body, 0)

    wait_scat(0)
    wait_scat(1)
    plsc.subcore_barrier()

    @pl.when(s < DRAIN_TILES)
    def _drain():
      pltpu.sync_copy(
          agg.at[pl.ds(s * DRAIN_ROWS, DRAIN_ROWS)],
          out_hbm.at[c, pl.ds(s * DRAIN_ROWS, DRAIN_ROWS)])

  return conv(x_src, row_idx, col_idx, q)


def _edge_mlp(edge_attr, We, be):
  BE = 3200

  def body(a_ref, w_ref, b_ref, o_ref):
    o_ref[...] = (
        jnp.dot(a_ref[...], w_ref[...], preferred_element_type=jnp.float32)
        + b_ref[...])

  return pl.pallas_call(
      body,
      grid=(E // BE,),
      in_specs=[
          pl.BlockSpec((BE, D_EDGE), lambda i: (i, 0)),
          pl.BlockSpec((D_EDGE, D), lambda i: (0, 0)),
          pl.BlockSpec((1, D), lambda i: (0, 0)),
      ],
      out_specs=pl.BlockSpec((BE, D), lambda i: (i, 0)),
      out_shape=jax.ShapeDtypeStruct((E, D), jnp.float32),
  )(edge_attr, We, be.reshape(1, D))


def _node_mlp(x_dst, aggs, W1, b1, W2, b2):
  BN = 2000

  def body(x_ref, a_ref, w1_ref, b1_ref, w2_ref, b2_ref, o_ref):
    h = x_ref[...] + a_ref[0] + a_ref[1]
    hid = jnp.maximum(
        jnp.dot(h, w1_ref[...], preferred_element_type=jnp.float32)
        + b1_ref[...], 0.0)
    o_ref[...] = (
        jnp.dot(hid, w2_ref[...], preferred_element_type=jnp.float32)
        + b2_ref[...])

  return pl.pallas_call(
      body,
      grid=(N_NODE // BN,),
      in_specs=[
          pl.BlockSpec((BN, D), lambda i: (i, 0)),
          pl.BlockSpec((NC, BN, D), lambda i: (0, i, 0)),
          pl.BlockSpec((D, D_HID), lambda i: (0, 0)),
          pl.BlockSpec((1, D_HID), lambda i: (0, 0)),
          pl.BlockSpec((D_HID, D), lambda i: (0, 0)),
          pl.BlockSpec((1, D), lambda i: (0, 0)),
      ],
      out_specs=pl.BlockSpec((BN, D), lambda i: (i, 0)),
      out_shape=jax.ShapeDtypeStruct((N_NODE, D), jnp.float32),
  )(x_dst, aggs, W1, b1.reshape(1, D_HID), W2, b2.reshape(1, D))


def kernel(cons_embedding, vals_embedding, v2c_edge_index, c2v_edge_index,
           v2c_edge_attr, c2v_edge_attr, cons_batch, vals_batch,
           v2c_We, v2c_be, v2c_W1, v2c_b1, v2c_W2, v2c_b2,
           c2v_We, c2v_be, c2v_W1, c2v_b1, c2v_W2, c2v_b2):
  row1 = v2c_edge_index[0].astype(jnp.int32)
  col1 = v2c_edge_index[1].astype(jnp.int32)
  row2 = c2v_edge_index[0].astype(jnp.int32)
  col2 = c2v_edge_index[1].astype(jnp.int32)

  q1 = _edge_mlp(v2c_edge_attr, v2c_We, v2c_be)
  q2 = _edge_mlp(c2v_edge_attr, c2v_We, c2v_be)

  agg1 = _sc_gather_scatter(vals_embedding, row1, col1, q1)
  cons_new = _node_mlp(cons_embedding, agg1, v2c_W1, v2c_b1, v2c_W2, v2c_b2)

  agg2 = _sc_gather_scatter(cons_new, row2, col2, q2)
  vals_new = _node_mlp(vals_embedding, agg2, c2v_W1, c2v_b1, c2v_W2, c2v_b2)

  return (vals_new, cons_new)


# speedup vs baseline: 4.7144x; 1.0271x over previous
import functools

import jax
import jax.numpy as jnp
from jax import lax
from jax.experimental import pallas as pl
from jax.experimental.pallas import tpu as pltpu
from jax.experimental.pallas import tpu_sc as plsc

N_NODE = 10000
E = 320000
D = 128
D_EDGE = 16
D_HID = 256

NC = 2
NS = 16
NW = NC * NS
EPW = E // NW
CHUNK = 80
NCHUNK = EPW // CHUNK
DRAIN_TILES = 10
DRAIN_ROWS = N_NODE // DRAIN_TILES
ZROWS = 40
LANES = 16


def _sc_gather_scatter(x_src, row_idx, col_idx, q):
  mesh = plsc.VectorSubcoreMesh(core_axis_name="c", subcore_axis_name="s",
                                num_cores=NC, num_subcores=NS)

  @functools.partial(
      pl.kernel,
      out_type=jax.ShapeDtypeStruct((NC, N_NODE, D), jnp.float32),
      mesh=mesh,
      scratch_types=[
          pltpu.VMEM((3, CHUNK), jnp.int32),
          pltpu.VMEM((3, CHUNK), jnp.int32),
          pltpu.VMEM((2, CHUNK, D), jnp.float32),
          pltpu.VMEM((2, CHUNK, D), jnp.float32),
          pltpu.VMEM_SHARED((N_NODE, D), jnp.float32),
          pltpu.SemaphoreType.DMA((3,)),
          pltpu.SemaphoreType.DMA((2,)),
          pltpu.SemaphoreType.DMA((2,)),
      ],
  )
  def conv(x_hbm, row_hbm, col_hbm, q_hbm, out_hbm, rowv, colv, qv, rv,
           agg, sidx, sem, sscat):
    c = lax.axis_index("c")
    s = lax.axis_index("s")
    wid = c * NS + s

    zero = jnp.zeros((LANES,), jnp.float32)

    def zbody(r, carry):
      for j in range(D // LANES):
        rv[0, r, pl.ds(j * LANES, LANES)] = zero
      return carry

    lax.fori_loop(0, ZROWS, zbody, 0)

    @pl.when(s < DRAIN_TILES)
    def _zero():
      for k in range(DRAIN_ROWS // ZROWS):
        pltpu.sync_copy(rv.at[0, pl.ds(0, ZROWS)],
                        agg.at[pl.ds(s * DRAIN_ROWS + k * ZROWS, ZROWS)])

    plsc.subcore_barrier()

    def fetch_idx(i, slot):
      base = wid * EPW + i * CHUNK
      pltpu.async_copy(row_hbm.at[pl.ds(base, CHUNK)], rowv.at[slot],
                       sidx.at[slot])
      pltpu.async_copy(col_hbm.at[pl.ds(base, CHUNK)], colv.at[slot],
                       sidx.at[slot])

    def wait_idx(slot):
      d = pltpu.make_async_copy(row_hbm.at[pl.ds(0, CHUNK)], rowv.at[slot],
                                sidx.at[slot])
      d.wait()
      d.wait()

    def fetch_q(i, dslot):
      base = wid * EPW + i * CHUNK
      pltpu.async_copy(q_hbm.at[pl.ds(base, CHUNK)], qv.at[dslot],
                       sem.at[dslot])

    def fetch_rows(islot, dslot):
      pltpu.async_copy(x_hbm.at[rowv.at[islot]], rv.at[dslot], sem.at[dslot])

    def fetch_data(i, islot, dslot):
      fetch_q(i, dslot)
      fetch_rows(islot, dslot)

    def wait_data(dslot):
      d = pltpu.make_async_copy(q_hbm.at[pl.ds(0, CHUNK)], qv.at[dslot],
                                sem.at[dslot])
      d.wait()
      d.wait()

    def wait_scat(dslot):
      pltpu.make_async_copy(rv.at[dslot], agg.at[pl.ds(0, CHUNK)],
                            sscat.at[dslot]).wait()

    fetch_idx(0, 0)
    fetch_idx(1, 1)
    wait_idx(0)
    fetch_data(0, 0, 0)

    def body(i, carry):
      i3 = lax.rem(i, 3)
      n3 = lax.rem(i + 1, 3)
      p3 = lax.rem(i + 2, 3)
      cur = lax.rem(i, 2)
      nxt = lax.rem(i + 1, 2)

      @pl.when(i + 1 < NCHUNK)
      def _launch_next():
        fetch_q(i + 1, nxt)
        wait_idx(n3)

        @pl.when(i >= 1)
        def _wait_prev_scatter():
          wait_scat(nxt)

        fetch_rows(n3, nxt)

      @pl.when(i + 2 < NCHUNK)
      def _prefetch_idx():
        fetch_idx(i + 2, p3)

      wait_data(cur)

      @plsc.parallel_loop(0, CHUNK, 1, unroll=8)
      def cbody(e):
        for j in range(D // LANES):
          sl = pl.ds(j * LANES, LANES)
          rv[cur, e, sl] = jnp.maximum(rv[cur, e, sl] + qv[cur, e, sl], 0.0)

      pltpu.async_copy(rv.at[cur], agg.at[colv.at[i3]], sscat.at[cur],
                       add=True)
      return carry

    lax.fori_loop(0, NCHUNK, body, 0)

    wait_scat(0)
    wait_scat(1)
    plsc.subcore_barrier()

    @pl.when(s < DRAIN_TILES)
    def _drain():
      pltpu.sync_copy(
          agg.at[pl.ds(s * DRAIN_ROWS, DRAIN_ROWS)],
          out_hbm.at[c, pl.ds(s * DRAIN_ROWS, DRAIN_ROWS)])

  return conv(x_src, row_idx, col_idx, q)


def _edge_mlp(edge_attr, We, be):
  BE = 3200

  def body(a_ref, w_ref, b_ref, o_ref):
    o_ref[...] = (
        jnp.dot(a_ref[...], w_ref[...], preferred_element_type=jnp.float32)
        + b_ref[...])

  return pl.pallas_call(
      body,
      grid=(E // BE,),
      in_specs=[
          pl.BlockSpec((BE, D_EDGE), lambda i: (i, 0)),
          pl.BlockSpec((D_EDGE, D), lambda i: (0, 0)),
          pl.BlockSpec((1, D), lambda i: (0, 0)),
      ],
      out_specs=pl.BlockSpec((BE, D), lambda i: (i, 0)),
      out_shape=jax.ShapeDtypeStruct((E, D), jnp.float32),
  )(edge_attr, We, be.reshape(1, D))


def _node_mlp(x_dst, aggs, W1, b1, W2, b2):
  BN = 2000

  def body(x_ref, a_ref, w1_ref, b1_ref, w2_ref, b2_ref, o_ref):
    h = x_ref[...] + a_ref[0] + a_ref[1]
    hid = jnp.maximum(
        jnp.dot(h, w1_ref[...], preferred_element_type=jnp.float32)
        + b1_ref[...], 0.0)
    o_ref[...] = (
        jnp.dot(hid, w2_ref[...], preferred_element_type=jnp.float32)
        + b2_ref[...])

  return pl.pallas_call(
      body,
      grid=(N_NODE // BN,),
      in_specs=[
          pl.BlockSpec((BN, D), lambda i: (i, 0)),
          pl.BlockSpec((NC, BN, D), lambda i: (0, i, 0)),
          pl.BlockSpec((D, D_HID), lambda i: (0, 0)),
          pl.BlockSpec((1, D_HID), lambda i: (0, 0)),
          pl.BlockSpec((D_HID, D), lambda i: (0, 0)),
          pl.BlockSpec((1, D), lambda i: (0, 0)),
      ],
      out_specs=pl.BlockSpec((BN, D), lambda i: (i, 0)),
      out_shape=jax.ShapeDtypeStruct((N_NODE, D), jnp.float32),
  )(x_dst, aggs, W1, b1.reshape(1, D_HID), W2, b2.reshape(1, D))


def kernel(cons_embedding, vals_embedding, v2c_edge_index, c2v_edge_index,
           v2c_edge_attr, c2v_edge_attr, cons_batch, vals_batch,
           v2c_We, v2c_be, v2c_W1, v2c_b1, v2c_W2, v2c_b2,
           c2v_We, c2v_be, c2v_W1, c2v_b1, c2v_W2, c2v_b2):
  row1 = v2c_edge_index[0].astype(jnp.int32)
  col1 = v2c_edge_index[1].astype(jnp.int32)
  row2 = c2v_edge_index[0].astype(jnp.int32)
  col2 = c2v_edge_index[1].astype(jnp.int32)

  q1 = _edge_mlp(v2c_edge_attr, v2c_We, v2c_be)
  q2 = _edge_mlp(c2v_edge_attr, c2v_We, c2v_be)

  agg1 = _sc_gather_scatter(vals_embedding, row1, col1, q1)
  cons_new = _node_mlp(cons_embedding, agg1, v2c_W1, v2c_b1, v2c_W2, v2c_b2)

  agg2 = _sc_gather_scatter(cons_new, row2, col2, q2)
  vals_new = _node_mlp(vals_embedding, agg2, c2v_W1, c2v_b1, c2v_W2, c2v_b2)

  return (vals_new, cons_new)


# speedup vs baseline: 4.9379x; 1.0474x over previous
import functools

import jax
import jax.numpy as jnp
from jax import lax
from jax.experimental import pallas as pl
from jax.experimental.pallas import tpu as pltpu
from jax.experimental.pallas import tpu_sc as plsc

N_NODE = 10000
E = 320000
D = 128
D_EDGE = 16
D_HID = 256

NC = 2
NS = 16
NW = NC * NS
EPW = E // NW
CHUNK = 80
NCHUNK = EPW // CHUNK
DRAIN_TILES = 10
DRAIN_ROWS = N_NODE // DRAIN_TILES
ZROWS = 40
LANES = 16


def _sc_gather_scatter(x_src, row_idx, col_idx, q):
  mesh = plsc.VectorSubcoreMesh(core_axis_name="c", subcore_axis_name="s",
                                num_cores=NC, num_subcores=NS)

  @functools.partial(
      pl.kernel,
      out_type=jax.ShapeDtypeStruct((NC, N_NODE, D), jnp.float32),
      mesh=mesh,
      scratch_types=[
          pltpu.VMEM((3, CHUNK), jnp.int32),
          pltpu.VMEM((3, CHUNK), jnp.int32),
          pltpu.VMEM((2, CHUNK, D), jnp.float32),
          pltpu.VMEM((2, CHUNK, D), jnp.float32),
          pltpu.VMEM_SHARED((N_NODE, D), jnp.float32),
          pltpu.SemaphoreType.DMA((3,)),
          pltpu.SemaphoreType.DMA((2,)),
          pltpu.SemaphoreType.DMA((2,)),
      ],
  )
  def conv(x_hbm, row_hbm, col_hbm, q_hbm, out_hbm, rowv, colv, qv, rv,
           agg, sidx, sem, sscat):
    c = lax.axis_index("c")
    s = lax.axis_index("s")
    wid = c * NS + s

    zero = jnp.zeros((LANES,), jnp.float32)

    def zbody(r, carry):
      for j in range(D // LANES):
        rv[0, r, pl.ds(j * LANES, LANES)] = zero
      return carry

    lax.fori_loop(0, ZROWS, zbody, 0)

    @pl.when(s < DRAIN_TILES)
    def _zero():
      for k in range(DRAIN_ROWS // ZROWS):
        pltpu.sync_copy(rv.at[0, pl.ds(0, ZROWS)],
                        agg.at[pl.ds(s * DRAIN_ROWS + k * ZROWS, ZROWS)])

    plsc.subcore_barrier()

    def fetch_idx(i, slot):
      base = wid * EPW + i * CHUNK
      pltpu.async_copy(row_hbm.at[pl.ds(base, CHUNK)], rowv.at[slot],
                       sidx.at[slot])
      pltpu.async_copy(col_hbm.at[pl.ds(base, CHUNK)], colv.at[slot],
                       sidx.at[slot])

    def wait_idx(slot):
      d = pltpu.make_async_copy(row_hbm.at[pl.ds(0, CHUNK)], rowv.at[slot],
                                sidx.at[slot])
      d.wait()
      d.wait()

    def fetch_q(i, dslot):
      base = wid * EPW + i * CHUNK
      pltpu.async_copy(q_hbm.at[pl.ds(base, CHUNK)], qv.at[dslot],
                       sem.at[dslot])

    def fetch_rows(islot, dslot):
      pltpu.async_copy(x_hbm.at[rowv.at[islot]], rv.at[dslot], sem.at[dslot])

    def fetch_data(i, islot, dslot):
      fetch_q(i, dslot)
      fetch_rows(islot, dslot)

    def wait_data(dslot):
      d = pltpu.make_async_copy(q_hbm.at[pl.ds(0, CHUNK)], qv.at[dslot],
                                sem.at[dslot])
      d.wait()
      d.wait()

    def wait_scat(dslot):
      pltpu.make_async_copy(rv.at[dslot], agg.at[pl.ds(0, CHUNK)],
                            sscat.at[dslot]).wait()

    fetch_idx(0, 0)
    fetch_idx(1, 1)
    wait_idx(0)
    fetch_data(0, 0, 0)

    def body(i, carry):
      i3 = lax.rem(i, 3)
      n3 = lax.rem(i + 1, 3)
      p3 = lax.rem(i + 2, 3)
      cur = lax.rem(i, 2)
      nxt = lax.rem(i + 1, 2)

      @pl.when(i + 1 < NCHUNK)
      def _launch_next():
        fetch_q(i + 1, nxt)
        wait_idx(n3)

        @pl.when(i >= 1)
        def _wait_prev_scatter():
          wait_scat(nxt)

        fetch_rows(n3, nxt)

      @pl.when(i + 2 < NCHUNK)
      def _prefetch_idx():
        fetch_idx(i + 2, p3)

      wait_data(cur)

      @plsc.parallel_loop(0, CHUNK, 1, unroll=16)
      def cbody(e):
        for j in range(D // LANES):
          sl = pl.ds(j * LANES, LANES)
          rv[cur, e, sl] = jnp.maximum(rv[cur, e, sl] + qv[cur, e, sl], 0.0)

      pltpu.async_copy(rv.at[cur], agg.at[colv.at[i3]], sscat.at[cur],
                       add=True)
      return carry

    lax.fori_loop(0, NCHUNK, body, 0)

    wait_scat(0)
    wait_scat(1)
    plsc.subcore_barrier()

    @pl.when(s < DRAIN_TILES)
    def _drain():
      pltpu.sync_copy(
          agg.at[pl.ds(s * DRAIN_ROWS, DRAIN_ROWS)],
          out_hbm.at[c, pl.ds(s * DRAIN_ROWS, DRAIN_ROWS)])

  return conv(x_src, row_idx, col_idx, q)


def _edge_mlp(edge_attr, We, be):
  BE = 6400

  def body(a_ref, w_ref, b_ref, o_ref):
    o_ref[...] = (
        jnp.dot(a_ref[...], w_ref[...], preferred_element_type=jnp.float32)
        + b_ref[...])

  return pl.pallas_call(
      body,
      grid=(E // BE,),
      in_specs=[
          pl.BlockSpec((BE, D_EDGE), lambda i: (i, 0)),
          pl.BlockSpec((D_EDGE, D), lambda i: (0, 0)),
          pl.BlockSpec((1, D), lambda i: (0, 0)),
      ],
      out_specs=pl.BlockSpec((BE, D), lambda i: (i, 0)),
      out_shape=jax.ShapeDtypeStruct((E, D), jnp.float32),
  )(edge_attr, We, be.reshape(1, D))


def _node_mlp(x_dst, aggs, W1, b1, W2, b2):
  BN = 2000

  def body(x_ref, a_ref, w1_ref, b1_ref, w2_ref, b2_ref, o_ref):
    h = x_ref[...] + a_ref[0] + a_ref[1]
    hid = jnp.maximum(
        jnp.dot(h, w1_ref[...], preferred_element_type=jnp.float32)
        + b1_ref[...], 0.0)
    o_ref[...] = (
        jnp.dot(hid, w2_ref[...], preferred_element_type=jnp.float32)
        + b2_ref[...])

  return pl.pallas_call(
      body,
      grid=(N_NODE // BN,),
      in_specs=[
          pl.BlockSpec((BN, D), lambda i: (i, 0)),
          pl.BlockSpec((NC, BN, D), lambda i: (0, i, 0)),
          pl.BlockSpec((D, D_HID), lambda i: (0, 0)),
          pl.BlockSpec((1, D_HID), lambda i: (0, 0)),
          pl.BlockSpec((D_HID, D), lambda i: (0, 0)),
          pl.BlockSpec((1, D), lambda i: (0, 0)),
      ],
      out_specs=pl.BlockSpec((BN, D), lambda i: (i, 0)),
      out_shape=jax.ShapeDtypeStruct((N_NODE, D), jnp.float32),
  )(x_dst, aggs, W1, b1.reshape(1, D_HID), W2, b2.reshape(1, D))


def kernel(cons_embedding, vals_embedding, v2c_edge_index, c2v_edge_index,
           v2c_edge_attr, c2v_edge_attr, cons_batch, vals_batch,
           v2c_We, v2c_be, v2c_W1, v2c_b1, v2c_W2, v2c_b2,
           c2v_We, c2v_be, c2v_W1, c2v_b1, c2v_W2, c2v_b2):
  row1 = v2c_edge_index[0].astype(jnp.int32)
  col1 = v2c_edge_index[1].astype(jnp.int32)
  row2 = c2v_edge_index[0].astype(jnp.int32)
  col2 = c2v_edge_index[1].astype(jnp.int32)

  q1 = _edge_mlp(v2c_edge_attr, v2c_We, v2c_be)
  q2 = _edge_mlp(c2v_edge_attr, c2v_We, c2v_be)

  agg1 = _sc_gather_scatter(vals_embedding, row1, col1, q1)
  cons_new = _node_mlp(cons_embedding, agg1, v2c_W1, v2c_b1, v2c_W2, v2c_b2)

  agg2 = _sc_gather_scatter(cons_new, row2, col2, q2)
  vals_new = _node_mlp(vals_embedding, agg2, c2v_W1, c2v_b1, c2v_W2, c2v_b2)

  return (vals_new, cons_new)


# speedup vs baseline: 4.9628x; 1.0051x over previous
import functools

import jax
import jax.numpy as jnp
from jax import lax
from jax.experimental import pallas as pl
from jax.experimental.pallas import tpu as pltpu
from jax.experimental.pallas import tpu_sc as plsc

N_NODE = 10000
E = 320000
D = 128
D_EDGE = 16
D_HID = 256

NC = 2
NS = 16
NW = NC * NS
EPW = E // NW
CHUNK = 80
NCHUNK = EPW // CHUNK
DRAIN_TILES = 10
DRAIN_ROWS = N_NODE // DRAIN_TILES
ZROWS = 40
LANES = 16


def _sc_gather_scatter(x_src, row_idx, col_idx, q):
  mesh = plsc.VectorSubcoreMesh(core_axis_name="c", subcore_axis_name="s",
                                num_cores=NC, num_subcores=NS)

  @functools.partial(
      pl.kernel,
      out_type=jax.ShapeDtypeStruct((NC, N_NODE, D), jnp.float32),
      mesh=mesh,
      scratch_types=[
          pltpu.VMEM((3, CHUNK), jnp.int32),
          pltpu.VMEM((3, CHUNK), jnp.int32),
          pltpu.VMEM((2, CHUNK, D), jnp.float32),
          pltpu.VMEM((2, CHUNK, D), jnp.float32),
          pltpu.VMEM_SHARED((N_NODE, D), jnp.float32),
          pltpu.SemaphoreType.DMA((3,)),
          pltpu.SemaphoreType.DMA((2,)),
          pltpu.SemaphoreType.DMA((2,)),
      ],
  )
  def conv(x_hbm, row_hbm, col_hbm, q_hbm, out_hbm, rowv, colv, qv, rv,
           agg, sidx, sem, sscat):
    c = lax.axis_index("c")
    s = lax.axis_index("s")
    wid = c * NS + s

    zero = jnp.zeros((LANES,), jnp.float32)

    def zbody(r, carry):
      for j in range(D // LANES):
        rv[0, r, pl.ds(j * LANES, LANES)] = zero
      return carry

    lax.fori_loop(0, ZROWS, zbody, 0)

    @pl.when(s < DRAIN_TILES)
    def _zero():
      for k in range(DRAIN_ROWS // ZROWS):
        pltpu.sync_copy(rv.at[0, pl.ds(0, ZROWS)],
                        agg.at[pl.ds(s * DRAIN_ROWS + k * ZROWS, ZROWS)])

    plsc.subcore_barrier()

    def fetch_idx(i, slot):
      base = wid * EPW + i * CHUNK
      pltpu.async_copy(row_hbm.at[pl.ds(base, CHUNK)], rowv.at[slot],
                       sidx.at[slot])
      pltpu.async_copy(col_hbm.at[pl.ds(base, CHUNK)], colv.at[slot],
                       sidx.at[slot])

    def wait_idx(slot):
      d = pltpu.make_async_copy(row_hbm.at[pl.ds(0, CHUNK)], rowv.at[slot],
                                sidx.at[slot])
      d.wait()
      d.wait()

    def fetch_q(i, dslot):
      base = wid * EPW + i * CHUNK
      pltpu.async_copy(q_hbm.at[pl.ds(base, CHUNK)], qv.at[dslot],
                       sem.at[dslot])

    def fetch_rows(islot, dslot):
      pltpu.async_copy(x_hbm.at[rowv.at[islot]], rv.at[dslot], sem.at[dslot])

    def fetch_data(i, islot, dslot):
      fetch_q(i, dslot)
      fetch_rows(islot, dslot)

    def wait_data(dslot):
      d = pltpu.make_async_copy(q_hbm.at[pl.ds(0, CHUNK)], qv.at[dslot],
                                sem.at[dslot])
      d.wait()
      d.wait()

    def wait_scat(dslot):
      pltpu.make_async_copy(rv.at[dslot], agg.at[pl.ds(0, CHUNK)],
                            sscat.at[dslot]).wait()

    fetch_idx(0, 0)
    fetch_idx(1, 1)
    wait_idx(0)
    fetch_data(0, 0, 0)

    def body(i, carry):
      i3 = lax.rem(i, 3)
      n3 = lax.rem(i + 1, 3)
      p3 = lax.rem(i + 2, 3)
      cur = lax.rem(i, 2)
      nxt = lax.rem(i + 1, 2)

      @pl.when(i + 1 < NCHUNK)
      def _launch_next():
        fetch_q(i + 1, nxt)
        wait_idx(n3)

        @pl.when(i >= 1)
        def _wait_prev_scatter():
          wait_scat(nxt)

        fetch_rows(n3, nxt)

      @pl.when(i + 2 < NCHUNK)
      def _prefetch_idx():
        fetch_idx(i + 2, p3)

      wait_data(cur)

      @plsc.parallel_loop(0, CHUNK, 1, unroll=20)
      def cbody(e):
        for j in range(D // LANES):
          sl = pl.ds(j * LANES, LANES)
          rv[cur, e, sl] = jnp.maximum(rv[cur, e, sl] + qv[cur, e, sl], 0.0)

      pltpu.async_copy(rv.at[cur], agg.at[colv.at[i3]], sscat.at[cur],
                       add=True)
      return carry

    lax.fori_loop(0, NCHUNK, body, 0)

    wait_scat(0)
    wait_scat(1)
    plsc.subcore_barrier()

    @pl.when(s < DRAIN_TILES)
    def _drain():
      pltpu.sync_copy(
          agg.at[pl.ds(s * DRAIN_ROWS, DRAIN_ROWS)],
          out_hbm.at[c, pl.ds(s * DRAIN_ROWS, DRAIN_ROWS)])

  return conv(x_src, row_idx, col_idx, q)


def _edge_mlp(edge_attr, We, be):
  BE = 12800

  def body(a_ref, w_ref, b_ref, o_ref):
    o_ref[...] = (
        jnp.dot(a_ref[...], w_ref[...], preferred_element_type=jnp.float32)
        + b_ref[...])

  return pl.pallas_call(
      body,
      grid=(E // BE,),
      in_specs=[
          pl.BlockSpec((BE, D_EDGE), lambda i: (i, 0)),
          pl.BlockSpec((D_EDGE, D), lambda i: (0, 0)),
          pl.BlockSpec((1, D), lambda i: (0, 0)),
      ],
      out_specs=pl.BlockSpec((BE, D), lambda i: (i, 0)),
      out_shape=jax.ShapeDtypeStruct((E, D), jnp.float32),
  )(edge_attr, We, be.reshape(1, D))


def _node_mlp(x_dst, aggs, W1, b1, W2, b2):
  BN = 2000

  def body(x_ref, a_ref, w1_ref, b1_ref, w2_ref, b2_ref, o_ref):
    h = x_ref[...] + a_ref[0] + a_ref[1]
    hid = jnp.maximum(
        jnp.dot(h, w1_ref[...], preferred_element_type=jnp.float32)
        + b1_ref[...], 0.0)
    o_ref[...] = (
        jnp.dot(hid, w2_ref[...], preferred_element_type=jnp.float32)
        + b2_ref[...])

  return pl.pallas_call(
      body,
      grid=(N_NODE // BN,),
      in_specs=[
          pl.BlockSpec((BN, D), lambda i: (i, 0)),
          pl.BlockSpec((NC, BN, D), lambda i: (0, i, 0)),
          pl.BlockSpec((D, D_HID), lambda i: (0, 0)),
          pl.BlockSpec((1, D_HID), lambda i: (0, 0)),
          pl.BlockSpec((D_HID, D), lambda i: (0, 0)),
          pl.BlockSpec((1, D), lambda i: (0, 0)),
      ],
      out_specs=pl.BlockSpec((BN, D), lambda i: (i, 0)),
      out_shape=jax.ShapeDtypeStruct((N_NODE, D), jnp.float32),
  )(x_dst, aggs, W1, b1.reshape(1, D_HID), W2, b2.reshape(1, D))


def kernel(cons_embedding, vals_embedding, v2c_edge_index, c2v_edge_index,
           v2c_edge_attr, c2v_edge_attr, cons_batch, vals_batch,
           v2c_We, v2c_be, v2c_W1, v2c_b1, v2c_W2, v2c_b2,
           c2v_We, c2v_be, c2v_W1, c2v_b1, c2v_W2, c2v_b2):
  row1 = v2c_edge_index[0].astype(jnp.int32)
  col1 = v2c_edge_index[1].astype(jnp.int32)
  row2 = c2v_edge_index[0].astype(jnp.int32)
  col2 = c2v_edge_index[1].astype(jnp.int32)

  q1 = _edge_mlp(v2c_edge_attr, v2c_We, v2c_be)
  q2 = _edge_mlp(c2v_edge_attr, c2v_We, c2v_be)

  agg1 = _sc_gather_scatter(vals_embedding, row1, col1, q1)
  cons_new = _node_mlp(cons_embedding, agg1, v2c_W1, v2c_b1, v2c_W2, v2c_b2)

  agg2 = _sc_gather_scatter(cons_new, row2, col2, q2)
  vals_new = _node_mlp(vals_embedding, agg2, c2v_W1, c2v_b1, c2v_W2, c2v_b2)

  return (vals_new, cons_new)


# speedup vs baseline: 4.9672x; 1.0009x over previous
import functools

import jax
import jax.numpy as jnp
from jax import lax
from jax.experimental import pallas as pl
from jax.experimental.pallas import tpu as pltpu
from jax.experimental.pallas import tpu_sc as plsc

N_NODE = 10000
E = 320000
D = 128
D_EDGE = 16
D_HID = 256

NC = 2
NS = 16
NW = NC * NS
EPW = E // NW
CHUNK = 80
NCHUNK = EPW // CHUNK
DRAIN_TILES = 10
DRAIN_ROWS = N_NODE // DRAIN_TILES
ZROWS = 40
LANES = 16


def _sc_gather_scatter(x_src, row_idx, col_idx, q):
  mesh = plsc.VectorSubcoreMesh(core_axis_name="c", subcore_axis_name="s",
                                num_cores=NC, num_subcores=NS)

  @functools.partial(
      pl.kernel,
      out_type=jax.ShapeDtypeStruct((NC, N_NODE, D), jnp.float32),
      mesh=mesh,
      scratch_types=[
          pltpu.VMEM((3, CHUNK), jnp.int32),
          pltpu.VMEM((3, CHUNK), jnp.int32),
          pltpu.VMEM((2, CHUNK, D), jnp.float32),
          pltpu.VMEM((2, CHUNK, D), jnp.float32),
          pltpu.VMEM_SHARED((N_NODE, D), jnp.float32),
          pltpu.SemaphoreType.DMA((3,)),
          pltpu.SemaphoreType.DMA((2,)),
          pltpu.SemaphoreType.DMA((2,)),
      ],
  )
  def conv(x_hbm, row_hbm, col_hbm, q_hbm, out_hbm, rowv, colv, qv, rv,
           agg, sidx, sem, sscat):
    c = lax.axis_index("c")
    s = lax.axis_index("s")
    wid = c * NS + s

    zero = jnp.zeros((LANES,), jnp.float32)

    def zbody(r, carry):
      for j in range(D // LANES):
        rv[0, r, pl.ds(j * LANES, LANES)] = zero
      return carry

    lax.fori_loop(0, ZROWS, zbody, 0)

    @pl.when(s < DRAIN_TILES)
    def _zero():
      for k in range(DRAIN_ROWS // ZROWS):
        pltpu.sync_copy(rv.at[0, pl.ds(0, ZROWS)],
                        agg.at[pl.ds(s * DRAIN_ROWS + k * ZROWS, ZROWS)])

    plsc.subcore_barrier()

    def fetch_idx(i, slot):
      base = wid * EPW + i * CHUNK
      pltpu.async_copy(row_hbm.at[pl.ds(base, CHUNK)], rowv.at[slot],
                       sidx.at[slot])
      pltpu.async_copy(col_hbm.at[pl.ds(base, CHUNK)], colv.at[slot],
                       sidx.at[slot])

    def wait_idx(slot):
      d = pltpu.make_async_copy(row_hbm.at[pl.ds(0, CHUNK)], rowv.at[slot],
                                sidx.at[slot])
      d.wait()
      d.wait()

    def fetch_q(i, dslot):
      base = wid * EPW + i * CHUNK
      pltpu.async_copy(q_hbm.at[pl.ds(base, CHUNK)], qv.at[dslot],
                       sem.at[dslot])

    def fetch_rows(islot, dslot):
      pltpu.async_copy(x_hbm.at[rowv.at[islot]], rv.at[dslot], sem.at[dslot])

    def fetch_data(i, islot, dslot):
      fetch_q(i, dslot)
      fetch_rows(islot, dslot)

    def wait_data(dslot):
      d = pltpu.make_async_copy(q_hbm.at[pl.ds(0, CHUNK)], qv.at[dslot],
                                sem.at[dslot])
      d.wait()
      d.wait()

    def wait_scat(dslot):
      pltpu.make_async_copy(rv.at[dslot], agg.at[pl.ds(0, CHUNK)],
                            sscat.at[dslot]).wait()

    fetch_idx(0, 0)
    fetch_idx(1, 1)
    wait_idx(0)
    fetch_data(0, 0, 0)

    def body(i, carry):
      i3 = lax.rem(i, 3)
      n3 = lax.rem(i + 1, 3)
      p3 = lax.rem(i + 2, 3)
      cur = lax.rem(i, 2)
      nxt = lax.rem(i + 1, 2)

      @pl.when(i + 1 < NCHUNK)
      def _launch_next():
        fetch_q(i + 1, nxt)
        wait_idx(n3)

        @pl.when(i >= 1)
        def _wait_prev_scatter():
          wait_scat(nxt)

        fetch_rows(n3, nxt)

      @pl.when(i + 2 < NCHUNK)
      def _prefetch_idx():
        fetch_idx(i + 2, p3)

      wait_data(cur)

      @plsc.parallel_loop(0, CHUNK, 1, unroll=40)
      def cbody(e):
        for j in range(D // LANES):
          sl = pl.ds(j * LANES, LANES)
          rv[cur, e, sl] = jnp.maximum(rv[cur, e, sl] + qv[cur, e, sl], 0.0)

      pltpu.async_copy(rv.at[cur], agg.at[colv.at[i3]], sscat.at[cur],
                       add=True)
      return carry

    lax.fori_loop(0, NCHUNK, body, 0)

    wait_scat(0)
    wait_scat(1)
    plsc.subcore_barrier()

    @pl.when(s < DRAIN_TILES)
    def _drain():
      pltpu.sync_copy(
          agg.at[pl.ds(s * DRAIN_ROWS, DRAIN_ROWS)],
          out_hbm.at[c, pl.ds(s * DRAIN_ROWS, DRAIN_ROWS)])

  return conv(x_src, row_idx, col_idx, q)


def _edge_mlp(edge_attr, We, be):
  BE = 12800

  def body(a_ref, w_ref, b_ref, o_ref):
    o_ref[...] = (
        jnp.dot(a_ref[...], w_ref[...], preferred_element_type=jnp.float32)
        + b_ref[...])

  return pl.pallas_call(
      body,
      grid=(E // BE,),
      in_specs=[
          pl.BlockSpec((BE, D_EDGE), lambda i: (i, 0)),
          pl.BlockSpec((D_EDGE, D), lambda i: (0, 0)),
          pl.BlockSpec((1, D), lambda i: (0, 0)),
      ],
      out_specs=pl.BlockSpec((BE, D), lambda i: (i, 0)),
      out_shape=jax.ShapeDtypeStruct((E, D), jnp.float32),
  )(edge_attr, We, be.reshape(1, D))


def _node_mlp(x_dst, aggs, W1, b1, W2, b2):
  BN = 2000

  def body(x_ref, a_ref, w1_ref, b1_ref, w2_ref, b2_ref, o_ref):
    h = x_ref[...] + a_ref[0] + a_ref[1]
    hid = jnp.maximum(
        jnp.dot(h, w1_ref[...], preferred_element_type=jnp.float32)
        + b1_ref[...], 0.0)
    o_ref[...] = (
        jnp.dot(hid, w2_ref[...], preferred_element_type=jnp.float32)
        + b2_ref[...])

  return pl.pallas_call(
      body,
      grid=(N_NODE // BN,),
      in_specs=[
          pl.BlockSpec((BN, D), lambda i: (i, 0)),
          pl.BlockSpec((NC, BN, D), lambda i: (0, i, 0)),
          pl.BlockSpec((D, D_HID), lambda i: (0, 0)),
          pl.BlockSpec((1, D_HID), lambda i: (0, 0)),
          pl.BlockSpec((D_HID, D), lambda i: (0, 0)),
          pl.BlockSpec((1, D), lambda i: (0, 0)),
      ],
      out_specs=pl.BlockSpec((BN, D), lambda i: (i, 0)),
      out_shape=jax.ShapeDtypeStruct((N_NODE, D), jnp.float32),
  )(x_dst, aggs, W1, b1.reshape(1, D_HID), W2, b2.reshape(1, D))


def kernel(cons_embedding, vals_embedding, v2c_edge_index, c2v_edge_index,
           v2c_edge_attr, c2v_edge_attr, cons_batch, vals_batch,
           v2c_We, v2c_be, v2c_W1, v2c_b1, v2c_W2, v2c_b2,
           c2v_We, c2v_be, c2v_W1, c2v_b1, c2v_W2, c2v_b2):
  row1 = v2c_edge_index[0].astype(jnp.int32)
  col1 = v2c_edge_index[1].astype(jnp.int32)
  row2 = c2v_edge_index[0].astype(jnp.int32)
  col2 = c2v_edge_index[1].astype(jnp.int32)

  q1 = _edge_mlp(v2c_edge_attr, v2c_We, v2c_be)
  q2 = _edge_mlp(c2v_edge_attr, c2v_We, c2v_be)

  agg1 = _sc_gather_scatter(vals_embedding, row1, col1, q1)
  cons_new = _node_mlp(cons_embedding, agg1, v2c_W1, v2c_b1, v2c_W2, v2c_b2)

  agg2 = _sc_gather_scatter(cons_new, row2, col2, q2)
  vals_new = _node_mlp(vals_embedding, agg2, c2v_W1, c2v_b1, c2v_W2, c2v_b2)

  return (vals_new, cons_new)
